# baseline (speedup 1.0000x reference)
import jax
import jax.numpy as jnp
from jax.experimental import pallas as pl
from jax.experimental.pallas import tpu as pltpu

EMBED_DIM = 64
FFN_DIM = 128
NUM_EXPERTS = 4


def _moe_kernel(x_ref, wg_ref, w1_ref, b1_ref, w2t_ref, b2_ref, o_ref,
                wgs, w1s, w2s, b1s, b2s):
    D, F, E = EMBED_DIM, FFN_DIM, NUM_EXPERTS

    @pl.when(pl.program_id(0) == 0)
    def _prep():
        wgs[:] = jnp.transpose(wg_ref[:], (1, 0))
        for e in range(E):
            w1s[e * F:(e + 1) * F, :] = jnp.transpose(w1_ref[e], (1, 0))
            w2s[e * D:(e + 1) * D, :] = w2t_ref[e]
            b1s[e * F:(e + 1) * F, 0:1] = jnp.transpose(b1_ref[e:e + 1, :],
                                                        (1, 0))
            b2s[e * D:(e + 1) * D, 0:1] = jnp.transpose(b2_ref[e:e + 1, :],
                                                        (1, 0))

    NSUB = 16
    HALF = x_ref.shape[1] // NSUB
    for h in range(NSUB):
        sl = pl.ds(h * HALF, HALF)
        xb = x_ref[:, sl]
        lT = jax.lax.dot_general(
            wgs[:], xb, (((1,), (0,)), ((), ())),
            preferred_element_type=jnp.float32)

        e_iota = jax.lax.broadcasted_iota(jnp.int32, lT.shape, 0)
        m1 = jnp.max(lT, axis=0, keepdims=True)
        idx1 = jnp.min(jnp.where(lT == m1, e_iota, E), axis=0, keepdims=True)
        masked = jnp.where(e_iota == idx1, -jnp.inf, lT)
        m2 = jnp.max(masked, axis=0, keepdims=True)
        idx2 = jnp.min(jnp.where(masked == m2, e_iota, E),
                       axis=0, keepdims=True)
        p1 = 1.0 / (1.0 + jnp.exp(m2 - m1))
        p2 = 1.0 - p1
        wT = (jnp.where(e_iota == idx1, p1, 0.0)
              + jnp.where(e_iota == idx2, p2, 0.0))

        hT = jax.lax.dot_general(
            w1s[:], xb, (((1,), (0,)), ((), ())),
            preferred_element_type=jnp.float32) + b1s[:]
        hT = jnp.maximum(hT, 0.0)

        acc = None
        for e in range(E):
            out_e = jax.lax.dot_general(
                w2s[e * D:(e + 1) * D, :], hT[e * F:(e + 1) * F, :],
                (((1,), (0,)), ((), ())),
                preferred_element_type=jnp.float32)
            term = wT[e:e + 1, :] * (out_e + b2s[e * D:(e + 1) * D, :])
            acc = term if acc is None else acc + term
        o_ref[:, sl] = acc


def kernel(x, Wg, W1, b1, W2, b2):
    x = x.reshape(-1, x.shape[-1])
    T, D = x.shape
    E, _, F = W1.shape
    xT = x.T
    W2t = W2.transpose(0, 2, 1)

    TB = 16384
    grid = (T // TB,)
    outT = pl.pallas_call(
        _moe_kernel,
        grid=grid,
        in_specs=[
            pl.BlockSpec((D, TB), lambda i: (0, i)),
            pl.BlockSpec((D, E), lambda i: (0, 0)),
            pl.BlockSpec((E, D, F), lambda i: (0, 0, 0)),
            pl.BlockSpec((E, F), lambda i: (0, 0)),
            pl.BlockSpec((E, D, F), lambda i: (0, 0, 0)),
            pl.BlockSpec((E, D), lambda i: (0, 0)),
        ],
        out_specs=pl.BlockSpec((D, TB), lambda i: (0, i)),
        out_shape=jax.ShapeDtypeStruct((D, T), jnp.float32),
        scratch_shapes=[
            pltpu.VMEM((E, D), jnp.float32),
            pltpu.VMEM((E * F, D), jnp.float32),
            pltpu.VMEM((E * D, F), jnp.float32),
            pltpu.VMEM((E * F, 1), jnp.float32),
            pltpu.VMEM((E * D, 1), jnp.float32),
        ],
        compiler_params=pltpu.CompilerParams(
            dimension_semantics=("arbitrary",)),
    )(xT, Wg, W1, b1, W2t, b2)
    return outT.T


# speedup vs baseline: 1.0404x; 1.0404x over previous
import jax
import jax.numpy as jnp
from jax.experimental import pallas as pl
from jax.experimental.pallas import tpu as pltpu

EMBED_DIM = 64
FFN_DIM = 128
NUM_EXPERTS = 4


def _moe_kernel(x_ref, wg_ref, w1_ref, b1_ref, w2t_ref, b2_ref, o_ref,
                wgs, w1s, w2s, b1s, b2s):
    D, F, E = EMBED_DIM, FFN_DIM, NUM_EXPERTS

    @pl.when(pl.program_id(0) == 0)
    def _prep():
        wgs[:] = jnp.transpose(wg_ref[:], (1, 0))
        for e in range(E):
            w1s[e * F:(e + 1) * F, :] = jnp.transpose(w1_ref[e], (1, 0))
            w2s[e * D:(e + 1) * D, :] = w2t_ref[e]
            b1s[e * F:(e + 1) * F, 0:1] = jnp.transpose(b1_ref[e:e + 1, :],
                                                        (1, 0))
            b2s[e * D:(e + 1) * D, 0:1] = jnp.transpose(b2_ref[e:e + 1, :],
                                                        (1, 0))

    NSUB = 8
    HALF = x_ref.shape[1] // NSUB
    for h in range(NSUB):
        sl = pl.ds(h * HALF, HALF)
        xb = x_ref[:, sl]
        lT = jax.lax.dot_general(
            wgs[:], xb, (((1,), (0,)), ((), ())),
            preferred_element_type=jnp.float32)

        e_iota = jax.lax.broadcasted_iota(jnp.int32, lT.shape, 0)
        m1 = jnp.max(lT, axis=0, keepdims=True)
        idx1 = jnp.min(jnp.where(lT == m1, e_iota, E), axis=0, keepdims=True)
        masked = jnp.where(e_iota == idx1, -jnp.inf, lT)
        m2 = jnp.max(masked, axis=0, keepdims=True)
        idx2 = jnp.min(jnp.where(masked == m2, e_iota, E),
                       axis=0, keepdims=True)
        p1 = 1.0 / (1.0 + jnp.exp(m2 - m1))
        p2 = 1.0 - p1
        wT = (jnp.where(e_iota == idx1, p1, 0.0)
              + jnp.where(e_iota == idx2, p2, 0.0))

        hT = jax.lax.dot_general(
            w1s[:], xb, (((1,), (0,)), ((), ())),
            preferred_element_type=jnp.float32) + b1s[:]
        hT = jnp.maximum(hT, 0.0)

        acc = None
        for e in range(E):
            out_e = jax.lax.dot_general(
                w2s[e * D:(e + 1) * D, :], hT[e * F:(e + 1) * F, :],
                (((1,), (0,)), ((), ())),
                preferred_element_type=jnp.float32)
            term = wT[e:e + 1, :] * (out_e + b2s[e * D:(e + 1) * D, :])
            acc = term if acc is None else acc + term
        o_ref[:, sl] = acc


def kernel(x, Wg, W1, b1, W2, b2):
    x = x.reshape(-1, x.shape[-1])
    T, D = x.shape
    E, _, F = W1.shape
    xT = x.T
    W2t = W2.transpose(0, 2, 1)

    TB = 8192
    grid = (T // TB,)
    outT = pl.pallas_call(
        _moe_kernel,
        grid=grid,
        in_specs=[
            pl.BlockSpec((D, TB), lambda i: (0, i)),
            pl.BlockSpec((D, E), lambda i: (0, 0)),
            pl.BlockSpec((E, D, F), lambda i: (0, 0, 0)),
            pl.BlockSpec((E, F), lambda i: (0, 0)),
            pl.BlockSpec((E, D, F), lambda i: (0, 0, 0)),
            pl.BlockSpec((E, D), lambda i: (0, 0)),
        ],
        out_specs=pl.BlockSpec((D, TB), lambda i: (0, i)),
        out_shape=jax.ShapeDtypeStruct((D, T), jnp.float32),
        scratch_shapes=[
            pltpu.VMEM((E, D), jnp.float32),
            pltpu.VMEM((E * F, D), jnp.float32),
            pltpu.VMEM((E * D, F), jnp.float32),
            pltpu.VMEM((E * F, 1), jnp.float32),
            pltpu.VMEM((E * D, 1), jnp.float32),
        ],
        compiler_params=pltpu.CompilerParams(
            dimension_semantics=("arbitrary",)),
    )(xT, Wg, W1, b1, W2t, b2)
    return outT.T
